# baseline (speedup 1.0000x reference)
import functools

import jax
import jax.numpy as jnp
from jax import lax
from jax.experimental import pallas as pl
from jax.experimental.pallas import tpu as pltpu
from jax.experimental.pallas import tpu_sc as plsc


def _build_adjacency(ei_flat, w, num_nodes):
    E = ei_flat.shape[0] // 2
    info = plsc.get_sparse_core_info()
    nc, ns = info.num_cores, info.num_subcores
    nwork = nc * ns
    rows = num_nodes // nwork
    blk = rows * num_nodes
    mesh = plsc.VectorSubcoreMesh(core_axis_name="c", subcore_axis_name="s")

    @functools.partial(
        pl.kernel,
        out_type=jax.ShapeDtypeStruct((num_nodes * num_nodes,), jnp.float32),
        mesh=mesh,
        compiler_params=pltpu.CompilerParams(needs_layout_passes=False),
        scratch_types=[
            pltpu.VMEM((E,), jnp.int32),
            pltpu.VMEM((E,), jnp.int32),
            pltpu.VMEM((E,), jnp.float32),
            pltpu.VMEM((blk,), jnp.float32),
        ],
    )
    def k(ei_hbm, w_hbm, out_hbm, src_v, dst_v, w_v, acc_v):
        wid = lax.axis_index("s") * nc + lax.axis_index("c")
        lo = wid * rows
        pltpu.sync_copy(ei_hbm.at[pl.ds(0, E)], src_v)
        pltpu.sync_copy(ei_hbm.at[pl.ds(E, E)], dst_v)
        pltpu.sync_copy(w_hbm, w_v)

        zero16 = jnp.zeros((16,), jnp.float32)

        def zrow(j, c):
            acc_v[pl.ds(j * 16, 16)] = zero16
            return c

        lax.fori_loop(0, blk // 16, zrow, 0, unroll=8)

        def body(i, carry):
            s16 = src_v[pl.ds(i * 16, 16)]
            d16 = dst_v[pl.ds(i * 16, 16)]
            w16 = w_v[pl.ds(i * 16, 16)]
            rel = s16 - lo
            m = (rel >= 0) & (rel < rows)
            relc = jnp.where(m, rel, 0)
            idx = relc * num_nodes + d16
            plsc.addupdate_scatter(acc_v, [idx], w16, mask=m)
            return carry

        lax.fori_loop(0, E // 16, body, 0, unroll=4)
        pltpu.sync_copy(acc_v, out_hbm.at[pl.ds(wid * blk, blk)])

    return k(ei_flat, w)


def _gnn_layers(x3, at, e1, benc_col, wbig, bl_cols, num_layers, bb):
    nblk, _, f = x3.shape
    c = e1.shape[0]

    def body(x_ref, at_ref, e1_ref, benc_ref, wbig_ref, blc_ref, out_ref):
        xb = x_ref[0]
        h = jnp.dot(e1_ref[...], xb, preferred_element_type=jnp.float32)
        h = h + benc_ref[...]
        for l in range(num_layers):
            hn = jnp.dot(wbig_ref[l], h, preferred_element_type=jnp.float32)
            hn = hn + blc_ref[l]
            msg = jnp.dot(hn, at_ref[...], preferred_element_type=jnp.float32)
            h = jnp.maximum(h + msg, 0.0)
        out_ref[...] = h

    return pl.pallas_call(
        body,
        grid=(nblk,),
        in_specs=[
            pl.BlockSpec((1, bb, f), lambda p: (p, 0, 0)),
            pl.BlockSpec((f, f), lambda p: (0, 0)),
            pl.BlockSpec((c, bb), lambda p: (0, 0)),
            pl.BlockSpec((c, 1), lambda p: (0, 0)),
            pl.BlockSpec((num_layers, c, c), lambda p: (0, 0, 0)),
            pl.BlockSpec((num_layers, c, 1), lambda p: (0, 0, 0)),
        ],
        out_specs=pl.BlockSpec((c, f), lambda p: (p, 0)),
        out_shape=jax.ShapeDtypeStruct((nblk * c, f), jnp.float32),
    )(x3, at, e1, benc_col, wbig, bl_cols)


def _transpose_wc1(wc1r, fc):
    o, f, h = wc1r.shape

    def body(in_ref, out_ref):
        out_ref[...] = jnp.transpose(in_ref[...], (2, 1, 0))

    return pl.pallas_call(
        body,
        grid=(f // fc,),
        in_specs=[pl.BlockSpec((o, fc, h), lambda p: (0, p, 0))],
        out_specs=pl.BlockSpec((h, fc, o), lambda p: (0, p, 0)),
        out_shape=jax.ShapeDtypeStruct((h, f, o), jnp.float32),
    )(wc1r)


def _classifier(h3, wc13, bc1_row, wc2t, bc2_row, hc):
    b, hh, f = h3.shape
    out = wc2t.shape[1]

    def body(h_ref, w_ref, bc1_ref, wc2_ref, bc2_ref, out_ref, acc):
        p = pl.program_id(0)

        @pl.when(p == 0)
        def _():
            acc[...] = jnp.zeros_like(acc)

        a = acc[...]
        for i in range(hc):
            a = a + jnp.dot(h_ref[:, i, :], w_ref[i],
                            preferred_element_type=jnp.float32)
        acc[...] = a

        @pl.when(p == pl.num_programs(0) - 1)
        def _():
            z = jnp.maximum(a + bc1_ref[...], 0.0)
            out_ref[...] = (
                jnp.dot(z, wc2_ref[...], preferred_element_type=jnp.float32)
                + bc2_ref[...]
            )

    return pl.pallas_call(
        body,
        grid=(hh // hc,),
        in_specs=[
            pl.BlockSpec((b, hc, f), lambda p: (0, p, 0)),
            pl.BlockSpec((hc, f, hh), lambda p: (p, 0, 0)),
            pl.BlockSpec((1, hh), lambda p: (0, 0)),
            pl.BlockSpec((hh, out), lambda p: (0, 0)),
            pl.BlockSpec((1, out), lambda p: (0, 0)),
        ],
        out_specs=pl.BlockSpec((b, out), lambda p: (0, 0)),
        out_shape=jax.ShapeDtypeStruct((b, out), jnp.float32),
        scratch_shapes=[pltpu.VMEM((b, hh), jnp.float32)],
    )(h3, wc13, bc1_row, wc2t, bc2_row)


def kernel(x, edge_index, edge_attr, w_enc, b_enc, Wls, bls, Wc1, bc1, Wc2, bc2):
    B, F = x.shape
    H = w_enc.shape[0]
    L = Wls.shape[0]
    OUT = Wc2.shape[0]
    BB = 4

    w = edge_attr.reshape(-1)

    at = _build_adjacency(edge_index.reshape(-1), w, F).reshape(F, F)

    eye = jnp.eye(BB, dtype=jnp.float32)
    e1 = jnp.kron(eye, w_enc)
    benc_col = jnp.tile(b_enc, BB)[:, None]
    wbig = jnp.stack([jnp.kron(eye, Wls[l]) for l in range(L)])
    bl_cols = jnp.tile(bls, (1, BB))[:, :, None]

    x3 = x.reshape(B // BB, BB, F)
    ht = _gnn_layers(x3, at, e1, benc_col, wbig, bl_cols, L, BB)

    wc13 = Wc1.reshape(H, F, H).transpose(2, 1, 0)
    h3 = ht.reshape(B, H, F)
    logits = _classifier(h3, wc13, bc1[None, :], Wc2.T, bc2[None, :], hc=8)
    return logits


# speedup vs baseline: 1.0198x; 1.0198x over previous
import functools

import jax
import jax.numpy as jnp
from jax import lax
from jax.experimental import pallas as pl
from jax.experimental.pallas import tpu as pltpu
from jax.experimental.pallas import tpu_sc as plsc


def _build_adjacency(ei_flat, w, num_nodes):
    E = ei_flat.shape[0] // 2
    info = plsc.get_sparse_core_info()
    nc, ns = info.num_cores, info.num_subcores
    nwork = nc * ns
    rows = num_nodes // nwork
    blk = rows * num_nodes
    mesh = plsc.VectorSubcoreMesh(core_axis_name="c", subcore_axis_name="s")

    @functools.partial(
        pl.kernel,
        out_type=jax.ShapeDtypeStruct((num_nodes * num_nodes,), jnp.float32),
        mesh=mesh,
        compiler_params=pltpu.CompilerParams(needs_layout_passes=False),
        scratch_types=[
            pltpu.VMEM((E,), jnp.int32),
            pltpu.VMEM((E,), jnp.int32),
            pltpu.VMEM((E,), jnp.float32),
            pltpu.VMEM((blk,), jnp.float32),
        ],
    )
    def k(ei_hbm, w_hbm, out_hbm, src_v, dst_v, w_v, acc_v):
        wid = lax.axis_index("s") * nc + lax.axis_index("c")
        lo = wid * rows
        pltpu.sync_copy(ei_hbm.at[pl.ds(0, E)], src_v)
        pltpu.sync_copy(ei_hbm.at[pl.ds(E, E)], dst_v)
        pltpu.sync_copy(w_hbm, w_v)

        zero16 = jnp.zeros((16,), jnp.float32)

        def zrow(j, c):
            acc_v[pl.ds(j * 16, 16)] = zero16
            return c

        lax.fori_loop(0, blk // 16, zrow, 0, unroll=8)

        def body(i, carry):
            s16 = src_v[pl.ds(i * 16, 16)]
            d16 = dst_v[pl.ds(i * 16, 16)]
            w16 = w_v[pl.ds(i * 16, 16)]
            rel = s16 - lo
            m = (rel >= 0) & (rel < rows)
            relc = jnp.where(m, rel, 0)
            idx = relc * num_nodes + d16
            plsc.addupdate_scatter(acc_v, [idx], w16, mask=m)
            return carry

        lax.fori_loop(0, E // 16, body, 0, unroll=4)
        pltpu.sync_copy(acc_v, out_hbm.at[pl.ds(wid * blk, blk)])

    return k(ei_flat, w)


def _gnn_layers(x3, at, wenc_col, benc_col, wls, bls, num_layers, bb):
    nblk, _, f = x3.shape
    hdim = wenc_col.shape[0]
    c = bb * hdim

    def body(x_ref, at_ref, wenc_ref, benc_ref, wls_ref, bls_ref, out_ref,
             wbig_s):
        p = pl.program_id(0)

        @pl.when(p == 0)
        def _():
            wbig_s[...] = jnp.zeros_like(wbig_s)
            for l in range(num_layers):
                wl = wls_ref[l]
                for i in range(bb):
                    wbig_s[l, pl.ds(i * hdim, hdim), pl.ds(i * hdim, hdim)] = wl

        xb = x_ref[0]
        h = (xb[:, None, :] * wenc_ref[...][None, :, :]
             + benc_ref[...][None, :, :]).reshape(c, f)
        for l in range(num_layers):
            hn = jnp.dot(wbig_s[l], h, preferred_element_type=jnp.float32)
            hn = hn + jnp.broadcast_to(bls_ref[l][None, :, :],
                                       (bb, hdim, 1)).reshape(c, 1)
            msg = jnp.dot(hn, at_ref[...], preferred_element_type=jnp.float32)
            h = jnp.maximum(h + msg, 0.0)
        out_ref[...] = h

    return pl.pallas_call(
        body,
        grid=(nblk,),
        in_specs=[
            pl.BlockSpec((1, bb, f), lambda p: (p, 0, 0)),
            pl.BlockSpec((f, f), lambda p: (0, 0)),
            pl.BlockSpec((hdim, 1), lambda p: (0, 0)),
            pl.BlockSpec((hdim, 1), lambda p: (0, 0)),
            pl.BlockSpec((num_layers, hdim, hdim), lambda p: (0, 0, 0)),
            pl.BlockSpec((num_layers, hdim, 1), lambda p: (0, 0, 0)),
        ],
        out_specs=pl.BlockSpec((c, f), lambda p: (p, 0)),
        out_shape=jax.ShapeDtypeStruct((nblk * c, f), jnp.float32),
        scratch_shapes=[pltpu.VMEM((num_layers, c, c), jnp.float32)],
    )(x3, at, wenc_col, benc_col, wls, bls)


def _classifier(h3, wc13, bc1_row, wc2t, bc2_row, hc):
    b, hh, f = h3.shape
    out = wc2t.shape[1]

    def body(h_ref, w_ref, bc1_ref, wc2_ref, bc2_ref, out_ref, acc):
        p = pl.program_id(0)

        @pl.when(p == 0)
        def _():
            acc[...] = jnp.zeros_like(acc)

        a = acc[...]
        for i in range(hc):
            a = a + jnp.dot(h_ref[:, i, :], w_ref[i],
                            preferred_element_type=jnp.float32)
        acc[...] = a

        @pl.when(p == pl.num_programs(0) - 1)
        def _():
            z = jnp.maximum(a + bc1_ref[...], 0.0)
            out_ref[...] = (
                jnp.dot(z, wc2_ref[...], preferred_element_type=jnp.float32)
                + bc2_ref[...]
            )

    return pl.pallas_call(
        body,
        grid=(hh // hc,),
        in_specs=[
            pl.BlockSpec((b, hc, f), lambda p: (0, p, 0)),
            pl.BlockSpec((hc, f, hh), lambda p: (p, 0, 0)),
            pl.BlockSpec((1, hh), lambda p: (0, 0)),
            pl.BlockSpec((hh, out), lambda p: (0, 0)),
            pl.BlockSpec((1, out), lambda p: (0, 0)),
        ],
        out_specs=pl.BlockSpec((b, out), lambda p: (0, 0)),
        out_shape=jax.ShapeDtypeStruct((b, out), jnp.float32),
        scratch_shapes=[pltpu.VMEM((b, hh), jnp.float32)],
    )(h3, wc13, bc1_row, wc2t, bc2_row)


def kernel(x, edge_index, edge_attr, w_enc, b_enc, Wls, bls, Wc1, bc1, Wc2, bc2):
    B, F = x.shape
    H = w_enc.shape[0]
    L = Wls.shape[0]
    OUT = Wc2.shape[0]
    BB = 4

    w = edge_attr.reshape(-1)

    at = _build_adjacency(edge_index.reshape(-1), w, F).reshape(F, F)

    x3 = x.reshape(B // BB, BB, F)
    ht = _gnn_layers(x3, at, w_enc, b_enc[:, None], Wls,
                     bls[:, :, None], L, BB)

    wc13 = Wc1.reshape(H, F, H).transpose(2, 1, 0)
    h3 = ht.reshape(B, H, F)
    logits = _classifier(h3, wc13, bc1[None, :], Wc2.T, bc2[None, :], hc=8)
    return logits


# speedup vs baseline: 1.0612x; 1.0405x over previous
import functools

import jax
import jax.numpy as jnp
from jax import lax
from jax.experimental import pallas as pl
from jax.experimental.pallas import tpu as pltpu
from jax.experimental.pallas import tpu_sc as plsc


def _build_adjacency(ei_flat, w, num_nodes):
    E = ei_flat.shape[0] // 2
    info = plsc.get_sparse_core_info()
    nc, ns = info.num_cores, info.num_subcores
    nwork = nc * ns
    rows = num_nodes // nwork
    blk = rows * num_nodes
    mesh = plsc.VectorSubcoreMesh(core_axis_name="c", subcore_axis_name="s")

    @functools.partial(
        pl.kernel,
        out_type=jax.ShapeDtypeStruct((num_nodes * num_nodes,), jnp.float32),
        mesh=mesh,
        compiler_params=pltpu.CompilerParams(needs_layout_passes=False),
        scratch_types=[
            pltpu.VMEM((E,), jnp.int32),
            pltpu.VMEM((E,), jnp.int32),
            pltpu.VMEM((E,), jnp.float32),
            pltpu.VMEM((blk,), jnp.float32),
        ],
    )
    def k(ei_hbm, w_hbm, out_hbm, src_v, dst_v, w_v, acc_v):
        wid = lax.axis_index("s") * nc + lax.axis_index("c")
        lo = wid * rows
        pltpu.sync_copy(ei_hbm.at[pl.ds(0, E)], src_v)
        pltpu.sync_copy(ei_hbm.at[pl.ds(E, E)], dst_v)
        pltpu.sync_copy(w_hbm, w_v)

        zero16 = jnp.zeros((16,), jnp.float32)

        def zrow(j, c):
            acc_v[pl.ds(j * 16, 16)] = zero16
            return c

        lax.fori_loop(0, blk // 16, zrow, 0, unroll=8)

        def body(i, carry):
            s16 = src_v[pl.ds(i * 16, 16)]
            d16 = dst_v[pl.ds(i * 16, 16)]
            w16 = w_v[pl.ds(i * 16, 16)]
            rel = s16 - lo
            m = (rel >= 0) & (rel < rows)
            relc = jnp.where(m, rel, 0)
            idx = relc * num_nodes + d16
            plsc.addupdate_scatter(acc_v, [idx], w16, mask=m)
            return carry

        lax.fori_loop(0, E // 16, body, 0, unroll=4)
        pltpu.sync_copy(acc_v, out_hbm.at[pl.ds(wid * blk, blk)])

    return k(ei_flat, w)


def _gnn_layers(x3, at, wenc_col, benc_col, wls, bls, num_layers, bb):
    nblk, _, f = x3.shape
    hdim = wenc_col.shape[0]
    c = bb * hdim
    nct = f // 128

    def body(x_ref, at_ref, wenc_ref, benc_ref, wls_ref, bls_ref, out_ref,
             wbig_s, at_s):
        p = pl.program_id(0)

        @pl.when(p == 0)
        def _():
            wbig_s[...] = jnp.zeros_like(wbig_s)
            for l in range(num_layers):
                wl = wls_ref[l]
                for i in range(bb):
                    wbig_s[l, pl.ds(i * hdim, hdim), pl.ds(i * hdim, hdim)] = wl
            for t in range(nct):
                at_s[:, pl.ds(t * 128, 128)] = at_ref[:, t, :]

        xb = x_ref[0]
        h = (xb[:, None, :] * wenc_ref[...][None, :, :]
             + benc_ref[...][None, :, :]).reshape(c, f)
        for l in range(num_layers):
            hn = jnp.dot(wbig_s[l], h, preferred_element_type=jnp.float32)
            hn = hn + jnp.broadcast_to(bls_ref[l][None, :, :],
                                       (bb, hdim, 1)).reshape(c, 1)
            msg = jnp.dot(hn, at_s[...], preferred_element_type=jnp.float32)
            h = jnp.maximum(h + msg, 0.0)
        out_ref[...] = h

    return pl.pallas_call(
        body,
        grid=(nblk,),
        in_specs=[
            pl.BlockSpec((1, bb, f), lambda p: (p, 0, 0)),
            pl.BlockSpec((f, nct, 128), lambda p: (0, 0, 0)),
            pl.BlockSpec((hdim, 1), lambda p: (0, 0)),
            pl.BlockSpec((hdim, 1), lambda p: (0, 0)),
            pl.BlockSpec((num_layers, hdim, hdim), lambda p: (0, 0, 0)),
            pl.BlockSpec((num_layers, hdim, 1), lambda p: (0, 0, 0)),
        ],
        out_specs=pl.BlockSpec((c, f), lambda p: (p, 0)),
        out_shape=jax.ShapeDtypeStruct((nblk * c, f), jnp.float32),
        scratch_shapes=[
            pltpu.VMEM((num_layers, c, c), jnp.float32),
            pltpu.VMEM((f, f), jnp.float32),
        ],
    )(x3, at, wenc_col, benc_col, wls, bls)


def _classifier(h3, wc13, bc1_row, wc2t, bc2_row, hc):
    b, hh, f = h3.shape
    out = wc2t.shape[1]

    def body(h_ref, w_ref, bc1_ref, wc2_ref, bc2_ref, out_ref, acc):
        p = pl.program_id(0)

        @pl.when(p == 0)
        def _():
            acc[...] = jnp.zeros_like(acc)

        a = acc[...]
        for i in range(hc):
            a = a + jnp.dot(h_ref[:, i, :], w_ref[i],
                            preferred_element_type=jnp.float32)
        acc[...] = a

        @pl.when(p == pl.num_programs(0) - 1)
        def _():
            z = jnp.maximum(a + bc1_ref[...], 0.0)
            out_ref[...] = (
                jnp.dot(z, wc2_ref[...], preferred_element_type=jnp.float32)
                + bc2_ref[...]
            )

    return pl.pallas_call(
        body,
        grid=(hh // hc,),
        in_specs=[
            pl.BlockSpec((b, hc, f), lambda p: (0, p, 0)),
            pl.BlockSpec((hc, f, hh), lambda p: (p, 0, 0)),
            pl.BlockSpec((1, hh), lambda p: (0, 0)),
            pl.BlockSpec((hh, out), lambda p: (0, 0)),
            pl.BlockSpec((1, out), lambda p: (0, 0)),
        ],
        out_specs=pl.BlockSpec((b, out), lambda p: (0, 0)),
        out_shape=jax.ShapeDtypeStruct((b, out), jnp.float32),
        scratch_shapes=[pltpu.VMEM((b, hh), jnp.float32)],
    )(h3, wc13, bc1_row, wc2t, bc2_row)


def kernel(x, edge_index, edge_attr, w_enc, b_enc, Wls, bls, Wc1, bc1, Wc2, bc2):
    B, F = x.shape
    H = w_enc.shape[0]
    L = Wls.shape[0]
    OUT = Wc2.shape[0]
    BB = 4

    w = edge_attr.reshape(-1)

    at = _build_adjacency(edge_index.reshape(-1), w, F).reshape(F, F // 128, 128)

    x3 = x.reshape(B // BB, BB, F)
    ht = _gnn_layers(x3, at, w_enc, b_enc[:, None], Wls,
                     bls[:, :, None], L, BB)

    wc13 = Wc1.reshape(H, F, H).transpose(2, 1, 0)
    h3 = ht.reshape(B, H, F)
    logits = _classifier(h3, wc13, bc1[None, :], Wc2.T, bc2[None, :], hc=8)
    return logits
